# baseline (speedup 1.0000x reference)
import jax
import jax.numpy as jnp
from jax.experimental import pallas as pl
from jax.experimental.pallas import tpu as pltpu


def _lstm_kernel(text_ref, e_hbm, u_ref, b_ref, v_hbm, wd_ref, bd_ref,
                 o_ref, v_vmem, xu_s, xbuf, h_ref, c_ref, sem_g, sem_v):
    t = pl.program_id(0)
    T = pl.num_programs(0)
    R = xbuf.shape[1]
    B = h_ref.shape[0]
    H = h_ref.shape[1]

    @pl.when(t == 0)
    def _():
        pltpu.make_async_copy(v_hbm, v_vmem, sem_v).start()
        for i in range(R):
            tok = text_ref[i]
            pltpu.make_async_copy(e_hbm.at[tok], xbuf.at[0, i],
                                  sem_g.at[0]).start()
        pltpu.make_async_copy(v_hbm, v_vmem, sem_v).wait()
        h_ref[...] = jnp.zeros_like(h_ref)
        c_ref[...] = jnp.zeros_like(c_ref)

    nxt = jnp.minimum(t + 1, T - 1)
    slot_n = (t + 1) % 2
    base = nxt * R

    slot = t % 2
    for i in range(R):
        pltpu.make_async_copy(e_hbm.at[0], xbuf.at[slot, i], sem_g.at[slot]).wait()

    x_bf = xbuf[slot].astype(jnp.bfloat16)
    xu_s[...] = (
        jnp.dot(x_bf, u_ref[...], preferred_element_type=jnp.float32)
        + b_ref[...]
    )

    h_new = h_ref[...]
    c_new = c_ref[...]
    n_steps = R // B
    chunk = R // n_steps
    for k in range(n_steps):
        for i in range(k * chunk, (k + 1) * chunk):
            tok = text_ref[base + i]
            pltpu.make_async_copy(e_hbm.at[tok], xbuf.at[slot_n, i],
                                  sem_g.at[slot_n]).start()
        gates = xu_s[k * B:(k + 1) * B, :] + jnp.dot(
            h_new.astype(jnp.bfloat16), v_vmem[...],
            preferred_element_type=jnp.float32)
        i_g = 0.5 * jnp.tanh(0.5 * gates[:, 0 * H:1 * H]) + 0.5
        f_g = 0.5 * jnp.tanh(0.5 * gates[:, 1 * H:2 * H]) + 0.5
        g_g = jnp.tanh(gates[:, 2 * H:3 * H])
        o_g = 0.5 * jnp.tanh(0.5 * gates[:, 3 * H:4 * H]) + 0.5
        c_new = f_g * c_new + i_g * g_g
        h_new = o_g * jnp.tanh(c_new)
    c_ref[...] = c_new
    h_ref[...] = h_new

    @pl.when(t == T - 1)
    def _():
        o_ref[...] = (
            jnp.dot(h_new, wd_ref[...], preferred_element_type=jnp.float32)
            + bd_ref[...]
        )
        for i in range(R):
            pltpu.make_async_copy(e_hbm.at[0], xbuf.at[0, i], sem_g.at[0]).wait()


def kernel(text, embed_table, U_i, U_f, U_c, U_o, V_i, V_f, V_c, V_o,
           b_i, b_f, b_c, b_o, W_dense, b_dense):
    VOCAB, E = embed_table.shape
    H = V_i.shape[0]
    B, S = text.shape
    H4 = 4 * H
    POL = W_dense.shape[1]

    U_cat = jnp.concatenate([U_i, U_f, U_c, U_o], axis=1).astype(jnp.bfloat16)
    b_cat = jnp.concatenate([b_i, b_f, b_c, b_o], axis=0).reshape(1, H4)
    V_cat = jnp.concatenate([V_i, V_f, V_c, V_o], axis=1).astype(jnp.bfloat16)
    text_t = text.T.astype(jnp.int32).reshape(S * B)
    Wp = jnp.pad(W_dense.astype(jnp.float32), ((0, 0), (0, 128 - POL)))
    bp = jnp.pad(b_dense.astype(jnp.float32), (0, 128 - POL)).reshape(1, 128)

    UNROLL = 16
    R = UNROLL * B
    out128 = pl.pallas_call(
        _lstm_kernel,
        out_shape=jax.ShapeDtypeStruct((B, 128), jnp.float32),
        grid=(S // UNROLL,),
        in_specs=[
            pl.BlockSpec(memory_space=pltpu.SMEM),
            pl.BlockSpec(memory_space=pl.ANY),
            pl.BlockSpec((E, H4), lambda t: (0, 0)),
            pl.BlockSpec((1, H4), lambda t: (0, 0)),
            pl.BlockSpec(memory_space=pl.ANY),
            pl.BlockSpec((H, 128), lambda t: (0, 0)),
            pl.BlockSpec((1, 128), lambda t: (0, 0)),
        ],
        out_specs=pl.BlockSpec((B, 128), lambda t: (0, 0)),
        scratch_shapes=[
            pltpu.VMEM((H, H4), jnp.bfloat16),
            pltpu.VMEM((R, H4), jnp.float32),
            pltpu.VMEM((2, R, E), jnp.float32),
            pltpu.VMEM((B, H), jnp.float32),
            pltpu.VMEM((B, H), jnp.float32),
            pltpu.SemaphoreType.DMA((2,)),
            pltpu.SemaphoreType.DMA,
        ],
        compiler_params=pltpu.CompilerParams(
            dimension_semantics=("arbitrary",),
        ),
        name="lstm_fused",
    )(text_t, embed_table, U_cat, b_cat, V_cat, Wp, bp)

    return out128[:, :POL]


# speedup vs baseline: 1.0064x; 1.0064x over previous
import jax
import jax.numpy as jnp
from jax.experimental import pallas as pl
from jax.experimental.pallas import tpu as pltpu


def _lstm_kernel(text_ref, e_hbm, u_ref, b_ref, v_hbm, wd_ref, bd_ref,
                 o_ref, v_vmem, xu_s, xbuf, h_ref, c_ref, sem_g, sem_v):
    t = pl.program_id(0)
    T = pl.num_programs(0)
    R = xbuf.shape[1]
    B = h_ref.shape[0]
    H = h_ref.shape[1]

    @pl.when(t == 0)
    def _():
        pltpu.make_async_copy(v_hbm, v_vmem, sem_v).start()
        for i in range(R):
            tok = text_ref[i]
            pltpu.make_async_copy(e_hbm.at[tok], xbuf.at[0, i],
                                  sem_g.at[0]).start()
        pltpu.make_async_copy(v_hbm, v_vmem, sem_v).wait()
        h_ref[...] = jnp.zeros_like(h_ref)
        c_ref[...] = jnp.zeros_like(c_ref)

    nxt = jnp.minimum(t + 1, T - 1)
    slot_n = (t + 1) % 2
    base = nxt * R

    slot = t % 2
    for i in range(R):
        pltpu.make_async_copy(e_hbm.at[0], xbuf.at[slot, i], sem_g.at[slot]).wait()

    x_bf = xbuf[slot].astype(jnp.bfloat16)
    xu_s[...] = (
        jnp.dot(x_bf, u_ref[...], preferred_element_type=jnp.float32)
        + b_ref[...]
    )

    h_new = h_ref[...]
    c_new = c_ref[...]
    n_steps = R // B
    chunk = R // n_steps
    for k in range(n_steps):
        for i in range(k * chunk, (k + 1) * chunk):
            tok = text_ref[base + i]
            pltpu.make_async_copy(e_hbm.at[tok], xbuf.at[slot_n, i],
                                  sem_g.at[slot_n]).start()
        gates = xu_s[k * B:(k + 1) * B, :] + jnp.dot(
            h_new.astype(jnp.bfloat16), v_vmem[...],
            preferred_element_type=jnp.float32)
        i_g = 0.5 * jnp.tanh(0.5 * gates[:, 0 * H:1 * H]) + 0.5
        f_g = 0.5 * jnp.tanh(0.5 * gates[:, 1 * H:2 * H]) + 0.5
        g_g = jnp.tanh(gates[:, 2 * H:3 * H])
        o_g = 0.5 * jnp.tanh(0.5 * gates[:, 3 * H:4 * H]) + 0.5
        c_new = f_g * c_new + i_g * g_g
        h_new = o_g * jnp.tanh(c_new)
    c_ref[...] = c_new
    h_ref[...] = h_new

    @pl.when(t == T - 1)
    def _():
        o_ref[...] = (
            jnp.dot(h_new, wd_ref[...], preferred_element_type=jnp.float32)
            + bd_ref[...]
        )
        for i in range(R):
            pltpu.make_async_copy(e_hbm.at[0], xbuf.at[0, i], sem_g.at[0]).wait()


def kernel(text, embed_table, U_i, U_f, U_c, U_o, V_i, V_f, V_c, V_o,
           b_i, b_f, b_c, b_o, W_dense, b_dense):
    VOCAB, E = embed_table.shape
    H = V_i.shape[0]
    B, S = text.shape
    H4 = 4 * H
    POL = W_dense.shape[1]

    U_cat = jnp.concatenate([U_i, U_f, U_c, U_o], axis=1).astype(jnp.bfloat16)
    b_cat = jnp.concatenate([b_i, b_f, b_c, b_o], axis=0).reshape(1, H4)
    V_cat = jnp.concatenate([V_i, V_f, V_c, V_o], axis=1).astype(jnp.bfloat16)
    text_t = text.T.astype(jnp.int32).reshape(S * B)
    Wp = jnp.pad(W_dense.astype(jnp.float32), ((0, 0), (0, 128 - POL)))
    bp = jnp.pad(b_dense.astype(jnp.float32), (0, 128 - POL)).reshape(1, 128)

    UNROLL = 8
    R = UNROLL * B
    out128 = pl.pallas_call(
        _lstm_kernel,
        out_shape=jax.ShapeDtypeStruct((B, 128), jnp.float32),
        grid=(S // UNROLL,),
        in_specs=[
            pl.BlockSpec(memory_space=pltpu.SMEM),
            pl.BlockSpec(memory_space=pl.ANY),
            pl.BlockSpec((E, H4), lambda t: (0, 0)),
            pl.BlockSpec((1, H4), lambda t: (0, 0)),
            pl.BlockSpec(memory_space=pl.ANY),
            pl.BlockSpec((H, 128), lambda t: (0, 0)),
            pl.BlockSpec((1, 128), lambda t: (0, 0)),
        ],
        out_specs=pl.BlockSpec((B, 128), lambda t: (0, 0)),
        scratch_shapes=[
            pltpu.VMEM((H, H4), jnp.bfloat16),
            pltpu.VMEM((R, H4), jnp.float32),
            pltpu.VMEM((2, R, E), jnp.float32),
            pltpu.VMEM((B, H), jnp.float32),
            pltpu.VMEM((B, H), jnp.float32),
            pltpu.SemaphoreType.DMA((2,)),
            pltpu.SemaphoreType.DMA,
        ],
        compiler_params=pltpu.CompilerParams(
            dimension_semantics=("arbitrary",),
        ),
        name="lstm_fused",
    )(text_t, embed_table, U_cat, b_cat, V_cat, Wp, bp)

    return out128[:, :POL]
